# baseline (speedup 1.0000x reference)
import jax
import jax.numpy as jnp
import numpy as np
from jax.experimental import pallas as pl
from jax.experimental.pallas import tpu as pltpu

_PH, _PW = 7, 7
_SMIN, _SMAX = 12, 29
_NSZ = _SMAX - _SMIN + 1
_CROP = 32
_N = 1000
_H = _W = 64
_C = 128


def _resize_table():
    mats = []
    for s in range(_SMIN, _SMAX + 1):
        eye = jnp.eye(s, dtype=jnp.float32)
        r = jax.image.resize(eye, (_PH, s), method="bilinear")
        r = jnp.pad(r, ((0, 8 - _PH), (0, _CROP - s)))
        mats.append(r)
    return jnp.stack(mats)


def _global_y_table(rtab):
    t = jnp.zeros((_NSZ, 32, 8, _H), jnp.float32)
    for y1 in range(32):
        t = t.at[:, y1, :, y1:y1 + _CROP].set(rtab)
    return t.reshape(_NSZ * 32, 8, _H)


def _floor_exact(a, b):
    s = a + b
    bb = s - a
    err = (a - (s - bb)) + (b - bb)
    fs = jnp.floor(s)
    return fs - jnp.where((s == fs) & (err < 0), 1.0, 0.0)


_G = 8


def _roi_kernel(rois_s, fm2_ref, wytab_ref, rtab_ref, out_ref):
    i = pl.program_id(0)
    for g in range(_G):
        r = i * _G + g
        x = rois_s[r, 0] * float(_W)
        y = rois_s[r, 1] * float(_H)
        w = rois_s[r, 2] * float(_W)
        h = rois_s[r, 3] * float(_H)
        x1 = jnp.floor(x).astype(jnp.int32)
        y1 = jnp.floor(y).astype(jnp.int32)
        kx = _floor_exact(x, w).astype(jnp.int32) - x1 - _SMIN
        ky = _floor_exact(y, h).astype(jnp.int32) - y1 - _SMIN

        wy = wytab_ref[ky * 32 + y1]
        rx = rtab_ref[kx]
        rhs = fm2_ref[:, pl.ds(x1 * _C, _CROP * _C)]

        a = jax.lax.dot_general(
            wy, rhs, (((1,), (0,)), ((), ())),
            preferred_element_type=jnp.float32,
        )
        a3 = a.reshape(8, _CROP, _C).astype(jnp.bfloat16)
        for p in range(_PH):
            op = jax.lax.dot_general(
                rx, a3[p], (((1,), (0,)), ((), ())),
                preferred_element_type=jnp.float32,
            )
            out_ref[g, p] = op[:_PW]


def kernel(feature_map, rois):
    fm2 = feature_map[0].reshape(_H, _W * _C).astype(jnp.bfloat16)
    rtab = _resize_table().astype(jnp.bfloat16)
    wytab = _global_y_table(_resize_table()).astype(jnp.bfloat16)
    grid_spec = pltpu.PrefetchScalarGridSpec(
        num_scalar_prefetch=1,
        grid=(_N // _G,),
        in_specs=[
            pl.BlockSpec((_H, _W * _C), lambda i, s: (0, 0)),
            pl.BlockSpec((_NSZ * 32, 8, _H), lambda i, s: (0, 0, 0)),
            pl.BlockSpec((_NSZ, 8, _CROP), lambda i, s: (0, 0, 0)),
        ],
        out_specs=pl.BlockSpec((_G, _PH, _PW, _C), lambda i, s: (i, 0, 0, 0)),
    )
    out = pl.pallas_call(
        _roi_kernel,
        grid_spec=grid_spec,
        out_shape=jax.ShapeDtypeStruct((_N, _PH, _PW, _C), jnp.float32),
    )(rois, fm2, wytab, rtab)
    return out


# speedup vs baseline: 1.0424x; 1.0424x over previous
import jax
import jax.numpy as jnp
import numpy as np
from jax.experimental import pallas as pl
from jax.experimental.pallas import tpu as pltpu

_PH, _PW = 7, 7
_SMIN, _SMAX = 12, 29
_NSZ = _SMAX - _SMIN + 1
_CROP = 32
_N = 1000
_H = _W = 64
_C = 128


def _resize_table():
    mats = []
    for s in range(_SMIN, _SMAX + 1):
        eye = jnp.eye(s, dtype=jnp.float32)
        r = jax.image.resize(eye, (_PH, s), method="bilinear")
        r = jnp.pad(r, ((0, 8 - _PH), (0, _CROP - s)))
        mats.append(r)
    return jnp.stack(mats)


def _global_y_table(rtab):
    t = jnp.zeros((_NSZ, 32, 8, _H), jnp.float32)
    for y1 in range(32):
        t = t.at[:, y1, :, y1:y1 + _CROP].set(rtab)
    return t.reshape(_NSZ * 32, 8, _H)


def _floor_exact(a, b):
    s = a + b
    bb = s - a
    err = (a - (s - bb)) + (b - bb)
    fs = jnp.floor(s)
    return fs - jnp.where((s == fs) & (err < 0), 1.0, 0.0)


_G = 8


def _roi_kernel(rois_s, fm2_ref, wytab_ref, rtab_ref, out_ref):
    i = pl.program_id(0)
    for g in range(_G):
        r = i * _G + g
        x = rois_s[r, 0] * float(_W)
        y = rois_s[r, 1] * float(_H)
        w = rois_s[r, 2] * float(_W)
        h = rois_s[r, 3] * float(_H)
        x1 = jnp.floor(x).astype(jnp.int32)
        y1 = jnp.floor(y).astype(jnp.int32)
        kx = _floor_exact(x, w).astype(jnp.int32) - x1 - _SMIN
        ky = _floor_exact(y, h).astype(jnp.int32) - y1 - _SMIN

        wy = wytab_ref[ky * 32 + y1]
        rx = rtab_ref[kx]
        rhs = fm2_ref[:, pl.ds(x1 * _C, _CROP * _C)]

        a = jax.lax.dot_general(
            wy, rhs, (((1,), (0,)), ((), ())),
            preferred_element_type=jnp.float32,
        )
        a3 = a.reshape(8, _CROP, _C)
        for p in range(_PH):
            op = jax.lax.dot_general(
                rx, a3[p], (((1,), (0,)), ((), ())),
                preferred_element_type=jnp.float32,
            )
            out_ref[g, p] = op[:_PW]


def kernel(feature_map, rois):
    fm2 = feature_map[0].reshape(_H, _W * _C).astype(jnp.bfloat16)
    rtab = _resize_table()
    wytab = _global_y_table(_resize_table()).astype(jnp.bfloat16)
    grid_spec = pltpu.PrefetchScalarGridSpec(
        num_scalar_prefetch=1,
        grid=(_N // _G,),
        in_specs=[
            pl.BlockSpec((_H, _W * _C), lambda i, s: (0, 0)),
            pl.BlockSpec((_NSZ * 32, 8, _H), lambda i, s: (0, 0, 0)),
            pl.BlockSpec((_NSZ, 8, _CROP), lambda i, s: (0, 0, 0)),
        ],
        out_specs=pl.BlockSpec((_G, _PH, _PW, _C), lambda i, s: (i, 0, 0, 0)),
    )
    out = pl.pallas_call(
        _roi_kernel,
        grid_spec=grid_spec,
        out_shape=jax.ShapeDtypeStruct((_N, _PH, _PW, _C), jnp.float32),
    )(rois, fm2, wytab, rtab)
    return out


# speedup vs baseline: 1.0738x; 1.0301x over previous
import jax
import jax.numpy as jnp
import numpy as np
from jax.experimental import pallas as pl
from jax.experimental.pallas import tpu as pltpu

_PH, _PW = 7, 7
_SMIN, _SMAX = 12, 29
_NSZ = _SMAX - _SMIN + 1
_CROP = 32
_N = 1000
_H = _W = 64
_C = 128


def _resize_table():
    mats = []
    for s in range(_SMIN, _SMAX + 1):
        eye = jnp.eye(s, dtype=jnp.float32)
        r = jax.image.resize(eye, (_PH, s), method="bilinear")
        r = jnp.pad(r, ((0, 8 - _PH), (0, _CROP - s)))
        mats.append(r)
    return jnp.stack(mats)


def _global_y_table(rtab):
    t = jnp.zeros((_NSZ, 32, 8, _H), jnp.float32)
    for y1 in range(32):
        t = t.at[:, y1, :, y1:y1 + _CROP].set(rtab)
    return t.reshape(_NSZ * 32, 8, _H)


def _floor_exact(a, b):
    s = a + b
    bb = s - a
    err = (a - (s - bb)) + (b - bb)
    fs = jnp.floor(s)
    return fs - jnp.where((s == fs) & (err < 0), 1.0, 0.0)


_G = 16


def _roi_kernel(rois_s, fm2_ref, wytab_ref, rtab_ref, out_ref):
    i = pl.program_id(0)
    for g in range(_G):
        r = i * _G + g
        x = rois_s[r, 0] * float(_W)
        y = rois_s[r, 1] * float(_H)
        w = rois_s[r, 2] * float(_W)
        h = rois_s[r, 3] * float(_H)
        x1 = jnp.floor(x).astype(jnp.int32)
        y1 = jnp.floor(y).astype(jnp.int32)
        kx = _floor_exact(x, w).astype(jnp.int32) - x1 - _SMIN
        ky = _floor_exact(y, h).astype(jnp.int32) - y1 - _SMIN

        wy = wytab_ref[ky * 32 + y1]
        rx = rtab_ref[kx]
        rhs = fm2_ref[:, pl.ds(x1 * _C, _CROP * _C)]

        a = jax.lax.dot_general(
            wy, rhs, (((1,), (0,)), ((), ())),
            preferred_element_type=jnp.float32,
        )
        a3 = a.reshape(8, _CROP, _C)
        for p in range(_PH):
            op = jax.lax.dot_general(
                rx, a3[p], (((1,), (0,)), ((), ())),
                preferred_element_type=jnp.float32,
            )
            out_ref[g, p] = op[:_PW]


def kernel(feature_map, rois):
    fm2 = feature_map[0].reshape(_H, _W * _C).astype(jnp.bfloat16)
    rtab = _resize_table()
    wytab = _global_y_table(_resize_table()).astype(jnp.bfloat16)
    grid_spec = pltpu.PrefetchScalarGridSpec(
        num_scalar_prefetch=1,
        grid=(_N // _G,),
        in_specs=[
            pl.BlockSpec((_H, _W * _C), lambda i, s: (0, 0)),
            pl.BlockSpec((_NSZ * 32, 8, _H), lambda i, s: (0, 0, 0)),
            pl.BlockSpec((_NSZ, 8, _CROP), lambda i, s: (0, 0, 0)),
        ],
        out_specs=pl.BlockSpec((_G, _PH, _PW, _C), lambda i, s: (i, 0, 0, 0)),
    )
    out = pl.pallas_call(
        _roi_kernel,
        grid_spec=grid_spec,
        out_shape=jax.ShapeDtypeStruct((_N, _PH, _PW, _C), jnp.float32),
    )(rois, fm2, wytab, rtab)
    return out
